# baseline (speedup 1.0000x reference)
import functools

import jax
import jax.numpy as jnp
from jax import lax
from jax.experimental import pallas as pl
from jax.experimental.pallas import tpu as pltpu
from jax.experimental.pallas import tpu_sc as plsc

N_NODES = 10000
N_EDGES = 320000
DIM_NODE = 128
DIM_EDGE = 16
HID = 2 * DIM_NODE
NEG_SLOPE = 0.2

_NC = 2
_NS = 16
_NW = _NC * _NS

_CHUNK = 128
_DEPTH = 4
_S = 2
_EH = N_EDGES // _S
_NCHUNK = 80
_BPW = _CHUNK * _NCHUNK
_B = _NW * _BPW


def _sc_gather(table, idx3):
    mesh = plsc.VectorSubcoreMesh(core_axis_name="c", subcore_axis_name="s")

    @functools.partial(
        pl.kernel,
        mesh=mesh,
        out_type=jax.ShapeDtypeStruct((_B, DIM_NODE), jnp.float32),
        scratch_types=(
            [pltpu.VMEM((_NCHUNK, _CHUNK), jnp.int32)]
            + [pltpu.VMEM((_CHUNK, DIM_NODE), jnp.float32)] * _DEPTH
            + [pltpu.SemaphoreType.DMA] * (2 * _DEPTH)
        ),
    )
    def gather_kernel(table_hbm, idx_hbm, out_hbm, idx_v, *bufs_and_sems):
        bufs = bufs_and_sems[:_DEPTH]
        gsems = bufs_and_sems[_DEPTH:2 * _DEPTH]
        ssems = bufs_and_sems[2 * _DEPTH:]
        wid = lax.axis_index("s") * _NC + lax.axis_index("c")
        pltpu.sync_copy(idx_hbm.at[wid], idx_v)

        def dst(c):
            return out_hbm.at[pl.ds((c * _NW + wid) * _CHUNK, _CHUNK)]

        def body(j, carry):
            for p in range(_DEPTH):
                c = _DEPTH * j + p

                @pl.when(j > 0)
                def _(p=p, c=c):
                    pltpu.make_async_copy(bufs[p], dst(c), ssems[p]).wait()

                pltpu.async_copy(table_hbm.at[idx_v.at[c]], bufs[p], gsems[p])
            for p in range(_DEPTH):
                c = _DEPTH * j + p
                pltpu.make_async_copy(
                    table_hbm.at[idx_v.at[c]], bufs[p], gsems[p]).wait()
                pltpu.async_copy(bufs[p], dst(c), ssems[p])
            return carry

        lax.fori_loop(0, _NCHUNK // _DEPTH, body, 0)
        tail = out_hbm.at[pl.ds(wid * _CHUNK, _CHUNK)]
        for p in range(_DEPTH):
            pltpu.make_async_copy(bufs[p], tail, ssems[p]).wait()

    return gather_kernel(table, idx3)


_E_BLK = 6400
_NB = _EH // _E_BLK
_OUT_ROWS = _E_BLK // 128


def _mlp_body(xi_ref, xj_ref, ea_ref, wa_ref, wb_ref, wc_ref, b1_ref, w2_ref,
              b2_ref, out_ref):
    h = jnp.dot(xi_ref[...], wa_ref[...], preferred_element_type=jnp.float32)
    h += jnp.dot(xj_ref[...], wb_ref[...], preferred_element_type=jnp.float32)
    h += jnp.dot(ea_ref[...], wc_ref[...], preferred_element_type=jnp.float32)
    h += b1_ref[...]
    h = jnp.where(h >= 0, h, NEG_SLOPE * h)
    res = jnp.sum(h * w2_ref[...], axis=1) + b2_ref[0, 0]
    out_ref[...] = res.reshape(1, _OUT_ROWS, 128)


def _tc_mlp(g, ea, wa, wb, wc, b1, w2, b2):
    return pl.pallas_call(
        _mlp_body,
        grid=(_NB,),
        in_specs=[
            pl.BlockSpec((_E_BLK, DIM_NODE), lambda i: (i, 0)),
            pl.BlockSpec((_E_BLK, DIM_NODE), lambda i: (i + _NB, 0)),
            pl.BlockSpec((_E_BLK, DIM_EDGE), lambda i: (i, 0)),
            pl.BlockSpec((DIM_NODE, HID), lambda i: (0, 0)),
            pl.BlockSpec((DIM_NODE, HID), lambda i: (0, 0)),
            pl.BlockSpec((DIM_EDGE, HID), lambda i: (0, 0)),
            pl.BlockSpec((1, HID), lambda i: (0, 0)),
            pl.BlockSpec((1, HID), lambda i: (0, 0)),
            pl.BlockSpec((1, 1), lambda i: (0, 0)),
        ],
        out_specs=pl.BlockSpec((1, _OUT_ROWS, 128), lambda i: (i, 0, 0)),
        out_shape=jax.ShapeDtypeStruct((_NB, _OUT_ROWS, 128), jnp.float32),
    )(g, g, ea, wa, wb, wc, b1, w2, b2)


def kernel(x, edge_index, edge_attr, W1, b1, W2, b2):
    src = edge_index[0, :].astype(jnp.int32)
    dst = edge_index[1, :].astype(jnp.int32)
    pad = jnp.arange(_B - 2 * _EH, dtype=jnp.int32) % N_NODES

    w1t = W1.T
    wa = w1t[:DIM_NODE]
    wb = w1t[DIM_NODE:2 * DIM_NODE]
    wc = w1t[2 * DIM_NODE:]
    b1r = b1.reshape(1, HID)
    w2r = W2.reshape(1, HID)
    b2r = b2.reshape(1, 1)

    outs = []
    for k in range(_S):
        sl = slice(k * _EH, (k + 1) * _EH)
        idx3 = (jnp.concatenate([src[sl], dst[sl], pad])
                .reshape(_NCHUNK, _NW, _CHUNK)
                .transpose(1, 0, 2))
        g = _sc_gather(x, idx3)
        outs.append(_tc_mlp(g, edge_attr[sl], wa, wb, wc, b1r, w2r, b2r))
    return jnp.concatenate(outs, axis=0).reshape(N_EDGES, 1)


# speedup vs baseline: 1.2021x; 1.2021x over previous
import functools

import jax
import jax.numpy as jnp
from jax import lax
from jax.experimental import pallas as pl
from jax.experimental.pallas import tpu as pltpu
from jax.experimental.pallas import tpu_sc as plsc

N_NODES = 10000
N_EDGES = 320000
DIM_NODE = 128
DIM_EDGE = 16
HID = 2 * DIM_NODE
NEG_SLOPE = 0.2

_NC = 2
_NS = 16
_NW = _NC * _NS

_CHUNK = 128
_DEPTH = 4
_S = 2
_EH = N_EDGES // _S
_NCHUNK = 80
_BPW = _CHUNK * _NCHUNK
_B = _NW * _BPW


def _sc_gather(table, idx3):
    mesh = plsc.VectorSubcoreMesh(core_axis_name="c", subcore_axis_name="s")

    @functools.partial(
        pl.kernel,
        mesh=mesh,
        out_type=jax.ShapeDtypeStruct((_B, DIM_NODE), jnp.float32),
        scratch_types=(
            [pltpu.VMEM((_NCHUNK, _CHUNK), jnp.int32)]
            + [pltpu.VMEM((_CHUNK, DIM_NODE), jnp.float32)] * _DEPTH
            + [pltpu.SemaphoreType.DMA] * (2 * _DEPTH)
        ),
    )
    def gather_kernel(table_hbm, idx_hbm, out_hbm, idx_v, *bufs_and_sems):
        bufs = bufs_and_sems[:_DEPTH]
        gsems = bufs_and_sems[_DEPTH:2 * _DEPTH]
        ssems = bufs_and_sems[2 * _DEPTH:]
        wid = lax.axis_index("s") * _NC + lax.axis_index("c")
        pltpu.sync_copy(idx_hbm.at[wid], idx_v)

        def dst(c):
            return out_hbm.at[pl.ds((c * _NW + wid) * _CHUNK, _CHUNK)]

        def body(j, carry):
            for p in range(_DEPTH):
                c = _DEPTH * j + p

                @pl.when(j > 0)
                def _(p=p, c=c):
                    pltpu.make_async_copy(bufs[p], dst(c), ssems[p]).wait()

                pltpu.async_copy(table_hbm.at[idx_v.at[c]], bufs[p], gsems[p])
            for p in range(_DEPTH):
                c = _DEPTH * j + p
                pltpu.make_async_copy(
                    table_hbm.at[idx_v.at[c]], bufs[p], gsems[p]).wait()
                pltpu.async_copy(bufs[p], dst(c), ssems[p])
            return carry

        lax.fori_loop(0, _NCHUNK // _DEPTH, body, 0)
        tail = out_hbm.at[pl.ds(wid * _CHUNK, _CHUNK)]
        for p in range(_DEPTH):
            pltpu.make_async_copy(bufs[p], tail, ssems[p]).wait()

    return gather_kernel(table, idx3)


_E_BLK = 6400
_NB = _EH // _E_BLK
_OUT_ROWS = _E_BLK // 128


def _mlp_body(xi_ref, xj_ref, eat_ref, wa_ref, wb_ref, wc_ref, b1_ref, w2_ref,
              b2_ref, out_ref):
    h = jnp.dot(xi_ref[...], wa_ref[...], preferred_element_type=jnp.float32)
    h += jnp.dot(xj_ref[...], wb_ref[...], preferred_element_type=jnp.float32)
    h += jax.lax.dot_general(
        eat_ref[...], wc_ref[...], (((0,), (0,)), ((), ())),
        preferred_element_type=jnp.float32)
    h += b1_ref[...]
    h = jnp.where(h >= 0, h, NEG_SLOPE * h)
    res = jnp.sum(h * w2_ref[...], axis=1) + b2_ref[0, 0]
    out_ref[...] = res.reshape(1, _OUT_ROWS, 128)


def _tc_mlp(g, ea, wa, wb, wc, b1, w2, b2):
    return pl.pallas_call(
        _mlp_body,
        grid=(_NB,),
        in_specs=[
            pl.BlockSpec((_E_BLK, DIM_NODE), lambda i: (i, 0)),
            pl.BlockSpec((_E_BLK, DIM_NODE), lambda i: (i + _NB, 0)),
            pl.BlockSpec((DIM_EDGE, _E_BLK), lambda i: (0, i)),
            pl.BlockSpec((DIM_NODE, HID), lambda i: (0, 0)),
            pl.BlockSpec((DIM_NODE, HID), lambda i: (0, 0)),
            pl.BlockSpec((DIM_EDGE, HID), lambda i: (0, 0)),
            pl.BlockSpec((1, HID), lambda i: (0, 0)),
            pl.BlockSpec((1, HID), lambda i: (0, 0)),
            pl.BlockSpec((1, 1), lambda i: (0, 0)),
        ],
        out_specs=pl.BlockSpec((1, _OUT_ROWS, 128), lambda i: (i, 0, 0)),
        out_shape=jax.ShapeDtypeStruct((_NB, _OUT_ROWS, 128), jnp.float32),
    )(g, g, ea, wa, wb, wc, b1, w2, b2)


def kernel(x, edge_index, edge_attr, W1, b1, W2, b2):
    src = edge_index[0, :].astype(jnp.int32)
    dst = edge_index[1, :].astype(jnp.int32)
    ea_t = edge_attr.T
    pad = jnp.arange(_B - 2 * _EH, dtype=jnp.int32) % N_NODES

    w1t = W1.T
    wa = w1t[:DIM_NODE]
    wb = w1t[DIM_NODE:2 * DIM_NODE]
    wc = w1t[2 * DIM_NODE:]
    b1r = b1.reshape(1, HID)
    w2r = W2.reshape(1, HID)
    b2r = b2.reshape(1, 1)

    outs = []
    for k in range(_S):
        sl = slice(k * _EH, (k + 1) * _EH)
        idx3 = (jnp.concatenate([src[sl], dst[sl], pad])
                .reshape(_NCHUNK, _NW, _CHUNK)
                .transpose(1, 0, 2))
        g = _sc_gather(x, idx3)
        outs.append(_tc_mlp(g, ea_t[:, sl], wa, wb, wc, b1r, w2r, b2r))
    return jnp.concatenate(outs, axis=0).reshape(N_EDGES, 1)


# speedup vs baseline: 1.2096x; 1.0062x over previous
import functools

import jax
import jax.numpy as jnp
from jax import lax
from jax.experimental import pallas as pl
from jax.experimental.pallas import tpu as pltpu
from jax.experimental.pallas import tpu_sc as plsc

N_NODES = 10000
N_EDGES = 320000
DIM_NODE = 128
DIM_EDGE = 16
HID = 2 * DIM_NODE
NEG_SLOPE = 0.2

_NC = 2
_NS = 16
_NW = _NC * _NS

_CHUNK = 128
_DEPTH = 4
_S = 1
_EH = N_EDGES // _S
_NCHUNK = 160
_BPW = _CHUNK * _NCHUNK
_B = _NW * _BPW


def _sc_gather(table, idx3):
    mesh = plsc.VectorSubcoreMesh(core_axis_name="c", subcore_axis_name="s")

    @functools.partial(
        pl.kernel,
        mesh=mesh,
        out_type=jax.ShapeDtypeStruct((_B, DIM_NODE), jnp.float32),
        scratch_types=(
            [pltpu.VMEM((_NCHUNK, _CHUNK), jnp.int32)]
            + [pltpu.VMEM((_CHUNK, DIM_NODE), jnp.float32)] * _DEPTH
            + [pltpu.SemaphoreType.DMA] * (2 * _DEPTH)
        ),
    )
    def gather_kernel(table_hbm, idx_hbm, out_hbm, idx_v, *bufs_and_sems):
        bufs = bufs_and_sems[:_DEPTH]
        gsems = bufs_and_sems[_DEPTH:2 * _DEPTH]
        ssems = bufs_and_sems[2 * _DEPTH:]
        wid = lax.axis_index("s") * _NC + lax.axis_index("c")
        pltpu.sync_copy(idx_hbm.at[wid], idx_v)

        def dst(c):
            return out_hbm.at[pl.ds((c * _NW + wid) * _CHUNK, _CHUNK)]

        def body(j, carry):
            for p in range(_DEPTH):
                c = _DEPTH * j + p

                @pl.when(j > 0)
                def _(p=p, c=c):
                    pltpu.make_async_copy(bufs[p], dst(c), ssems[p]).wait()

                pltpu.async_copy(table_hbm.at[idx_v.at[c]], bufs[p], gsems[p])
            for p in range(_DEPTH):
                c = _DEPTH * j + p
                pltpu.make_async_copy(
                    table_hbm.at[idx_v.at[c]], bufs[p], gsems[p]).wait()
                pltpu.async_copy(bufs[p], dst(c), ssems[p])
            return carry

        lax.fori_loop(0, _NCHUNK // _DEPTH, body, 0)
        tail = out_hbm.at[pl.ds(wid * _CHUNK, _CHUNK)]
        for p in range(_DEPTH):
            pltpu.make_async_copy(bufs[p], tail, ssems[p]).wait()

    return gather_kernel(table, idx3)


_E_BLK = 6400
_NB = _EH // _E_BLK
_OUT_ROWS = _E_BLK // 128


def _mlp_body(xi_ref, xj_ref, eat_ref, wa_ref, wb_ref, wc_ref, b1_ref, w2_ref,
              b2_ref, out_ref):
    h = jnp.dot(xi_ref[...], wa_ref[...], preferred_element_type=jnp.float32)
    h += jnp.dot(xj_ref[...], wb_ref[...], preferred_element_type=jnp.float32)
    h += jax.lax.dot_general(
        eat_ref[...], wc_ref[...], (((0,), (0,)), ((), ())),
        preferred_element_type=jnp.float32)
    h += b1_ref[...]
    h = jnp.where(h >= 0, h, NEG_SLOPE * h)
    res = jnp.sum(h * w2_ref[...], axis=1) + b2_ref[0, 0]
    out_ref[...] = res.reshape(1, _OUT_ROWS, 128)


def _tc_mlp(g, ea, wa, wb, wc, b1, w2, b2):
    return pl.pallas_call(
        _mlp_body,
        grid=(_NB,),
        in_specs=[
            pl.BlockSpec((_E_BLK, DIM_NODE), lambda i: (i, 0)),
            pl.BlockSpec((_E_BLK, DIM_NODE), lambda i: (i + _NB, 0)),
            pl.BlockSpec((DIM_EDGE, _E_BLK), lambda i: (0, i)),
            pl.BlockSpec((DIM_NODE, HID), lambda i: (0, 0)),
            pl.BlockSpec((DIM_NODE, HID), lambda i: (0, 0)),
            pl.BlockSpec((DIM_EDGE, HID), lambda i: (0, 0)),
            pl.BlockSpec((1, HID), lambda i: (0, 0)),
            pl.BlockSpec((1, HID), lambda i: (0, 0)),
            pl.BlockSpec((1, 1), lambda i: (0, 0)),
        ],
        out_specs=pl.BlockSpec((1, _OUT_ROWS, 128), lambda i: (i, 0, 0)),
        out_shape=jax.ShapeDtypeStruct((_NB, _OUT_ROWS, 128), jnp.float32),
    )(g, g, ea, wa, wb, wc, b1, w2, b2)


def kernel(x, edge_index, edge_attr, W1, b1, W2, b2):
    src = edge_index[0, :].astype(jnp.int32)
    dst = edge_index[1, :].astype(jnp.int32)
    ea_t = edge_attr.T
    pad = jnp.arange(_B - 2 * _EH, dtype=jnp.int32) % N_NODES

    w1t = W1.T
    wa = w1t[:DIM_NODE]
    wb = w1t[DIM_NODE:2 * DIM_NODE]
    wc = w1t[2 * DIM_NODE:]
    b1r = b1.reshape(1, HID)
    w2r = W2.reshape(1, HID)
    b2r = b2.reshape(1, 1)

    outs = []
    for k in range(_S):
        sl = slice(k * _EH, (k + 1) * _EH)
        idx3 = (jnp.concatenate([src[sl], dst[sl], pad])
                .reshape(_NCHUNK, _NW, _CHUNK)
                .transpose(1, 0, 2))
        g = _sc_gather(x, idx3)
        outs.append(_tc_mlp(g, ea_t[:, sl], wa, wb, wc, b1r, w2r, b2r))
    return jnp.concatenate(outs, axis=0).reshape(N_EDGES, 1)


# speedup vs baseline: 1.2175x; 1.0066x over previous
import functools

import jax
import jax.numpy as jnp
from jax import lax
from jax.experimental import pallas as pl
from jax.experimental.pallas import tpu as pltpu
from jax.experimental.pallas import tpu_sc as plsc

N_NODES = 10000
N_EDGES = 320000
DIM_NODE = 128
DIM_EDGE = 16
HID = 2 * DIM_NODE
NEG_SLOPE = 0.2

_NC = 2
_NS = 16
_NW = _NC * _NS

_CHUNK = 128
_DEPTH = 5
_S = 1
_EH = N_EDGES // _S
_NCHUNK = 160
_BPW = _CHUNK * _NCHUNK
_B = _NW * _BPW


def _sc_gather(table, idx3):
    mesh = plsc.VectorSubcoreMesh(core_axis_name="c", subcore_axis_name="s")

    @functools.partial(
        pl.kernel,
        mesh=mesh,
        out_type=jax.ShapeDtypeStruct((_B, DIM_NODE), jnp.float32),
        scratch_types=(
            [pltpu.VMEM((_NCHUNK, _CHUNK), jnp.int32)]
            + [pltpu.VMEM((_CHUNK, DIM_NODE), jnp.float32)] * _DEPTH
            + [pltpu.SemaphoreType.DMA] * (2 * _DEPTH)
        ),
    )
    def gather_kernel(table_hbm, idx_hbm, out_hbm, idx_v, *bufs_and_sems):
        bufs = bufs_and_sems[:_DEPTH]
        gsems = bufs_and_sems[_DEPTH:2 * _DEPTH]
        ssems = bufs_and_sems[2 * _DEPTH:]
        wid = lax.axis_index("s") * _NC + lax.axis_index("c")
        pltpu.sync_copy(idx_hbm.at[wid], idx_v)

        def dst(c):
            return out_hbm.at[pl.ds((c * _NW + wid) * _CHUNK, _CHUNK)]

        def body(j, carry):
            for p in range(_DEPTH):
                c = _DEPTH * j + p

                @pl.when(j > 0)
                def _(p=p, c=c):
                    pltpu.make_async_copy(bufs[p], dst(c), ssems[p]).wait()

                pltpu.async_copy(table_hbm.at[idx_v.at[c]], bufs[p], gsems[p])
            for p in range(_DEPTH):
                c = _DEPTH * j + p
                pltpu.make_async_copy(
                    table_hbm.at[idx_v.at[c]], bufs[p], gsems[p]).wait()
                pltpu.async_copy(bufs[p], dst(c), ssems[p])
            return carry

        lax.fori_loop(0, _NCHUNK // _DEPTH, body, 0)
        tail = out_hbm.at[pl.ds(wid * _CHUNK, _CHUNK)]
        for p in range(_DEPTH):
            pltpu.make_async_copy(bufs[p], tail, ssems[p]).wait()

    return gather_kernel(table, idx3)


_E_BLK = 6400
_NB = _EH // _E_BLK
_OUT_ROWS = _E_BLK // 128


def _mlp_body(xi_ref, xj_ref, eat_ref, wa_ref, wb_ref, wc_ref, b1_ref, w2_ref,
              b2_ref, out_ref):
    h = jnp.dot(xi_ref[...], wa_ref[...], preferred_element_type=jnp.float32)
    h += jnp.dot(xj_ref[...], wb_ref[...], preferred_element_type=jnp.float32)
    h += jax.lax.dot_general(
        eat_ref[...], wc_ref[...], (((0,), (0,)), ((), ())),
        preferred_element_type=jnp.float32)
    h += b1_ref[...]
    h = jnp.where(h >= 0, h, NEG_SLOPE * h)
    res = jnp.sum(h * w2_ref[...], axis=1) + b2_ref[0, 0]
    out_ref[...] = res.reshape(1, _OUT_ROWS, 128)


def _tc_mlp(g, ea, wa, wb, wc, b1, w2, b2):
    return pl.pallas_call(
        _mlp_body,
        grid=(_NB,),
        in_specs=[
            pl.BlockSpec((_E_BLK, DIM_NODE), lambda i: (i, 0)),
            pl.BlockSpec((_E_BLK, DIM_NODE), lambda i: (i + _NB, 0)),
            pl.BlockSpec((DIM_EDGE, _E_BLK), lambda i: (0, i)),
            pl.BlockSpec((DIM_NODE, HID), lambda i: (0, 0)),
            pl.BlockSpec((DIM_NODE, HID), lambda i: (0, 0)),
            pl.BlockSpec((DIM_EDGE, HID), lambda i: (0, 0)),
            pl.BlockSpec((1, HID), lambda i: (0, 0)),
            pl.BlockSpec((1, HID), lambda i: (0, 0)),
            pl.BlockSpec((1, 1), lambda i: (0, 0)),
        ],
        out_specs=pl.BlockSpec((1, _OUT_ROWS, 128), lambda i: (i, 0, 0)),
        out_shape=jax.ShapeDtypeStruct((_NB, _OUT_ROWS, 128), jnp.float32),
    )(g, g, ea, wa, wb, wc, b1, w2, b2)


def kernel(x, edge_index, edge_attr, W1, b1, W2, b2):
    src = edge_index[0, :].astype(jnp.int32)
    dst = edge_index[1, :].astype(jnp.int32)
    ea_t = edge_attr.T
    pad = jnp.arange(_B - 2 * _EH, dtype=jnp.int32) % N_NODES

    w1t = W1.T
    wa = w1t[:DIM_NODE]
    wb = w1t[DIM_NODE:2 * DIM_NODE]
    wc = w1t[2 * DIM_NODE:]
    b1r = b1.reshape(1, HID)
    w2r = W2.reshape(1, HID)
    b2r = b2.reshape(1, 1)

    outs = []
    for k in range(_S):
        sl = slice(k * _EH, (k + 1) * _EH)
        idx3 = (jnp.concatenate([src[sl], dst[sl], pad])
                .reshape(_NCHUNK, _NW, _CHUNK)
                .transpose(1, 0, 2))
        g = _sc_gather(x, idx3)
        outs.append(_tc_mlp(g, ea_t[:, sl], wa, wb, wc, b1r, w2r, b2r))
    return jnp.concatenate(outs, axis=0).reshape(N_EDGES, 1)


# speedup vs baseline: 1.2341x; 1.0136x over previous
import functools

import jax
import jax.numpy as jnp
from jax import lax
from jax.experimental import pallas as pl
from jax.experimental.pallas import tpu as pltpu
from jax.experimental.pallas import tpu_sc as plsc

N_NODES = 10000
N_EDGES = 320000
DIM_NODE = 128
DIM_EDGE = 16
HID = 2 * DIM_NODE
NEG_SLOPE = 0.2

_NC = 2
_NS = 16
_NW = _NC * _NS

_CHUNK = 128
_DEPTH = 5
_S = 1
_EH = N_EDGES // _S
_NCHUNK = 160
_BPW = _CHUNK * _NCHUNK
_B = _NW * _BPW


def _sc_gather(table, idx3):
    mesh = plsc.VectorSubcoreMesh(core_axis_name="c", subcore_axis_name="s")

    @functools.partial(
        pl.kernel,
        mesh=mesh,
        out_type=jax.ShapeDtypeStruct((_B, DIM_NODE), jnp.float32),
        scratch_types=(
            [pltpu.VMEM((_NCHUNK, _CHUNK), jnp.int32)]
            + [pltpu.VMEM((_CHUNK, DIM_NODE), jnp.float32)] * _DEPTH
            + [pltpu.SemaphoreType.DMA] * (2 * _DEPTH)
        ),
    )
    def gather_kernel(table_hbm, idx_hbm, out_hbm, idx_v, *bufs_and_sems):
        bufs = bufs_and_sems[:_DEPTH]
        gsems = bufs_and_sems[_DEPTH:2 * _DEPTH]
        ssems = bufs_and_sems[2 * _DEPTH:]
        wid = lax.axis_index("s") * _NC + lax.axis_index("c")
        pltpu.sync_copy(idx_hbm.at[wid], idx_v)

        def dst(c):
            return out_hbm.at[pl.ds((c * _NW + wid) * _CHUNK, _CHUNK)]

        def body(j, carry):
            for p in range(_DEPTH):
                c = _DEPTH * j + p

                @pl.when(j > 0)
                def _(p=p, c=c):
                    pltpu.make_async_copy(bufs[p], dst(c), ssems[p]).wait()

                pltpu.async_copy(table_hbm.at[idx_v.at[c]], bufs[p], gsems[p])
            for p in range(_DEPTH):
                c = _DEPTH * j + p
                pltpu.make_async_copy(
                    table_hbm.at[idx_v.at[c]], bufs[p], gsems[p]).wait()
                pltpu.async_copy(bufs[p], dst(c), ssems[p])
            return carry

        lax.fori_loop(0, _NCHUNK // _DEPTH, body, 0)
        tail = out_hbm.at[pl.ds(wid * _CHUNK, _CHUNK)]
        for p in range(_DEPTH):
            pltpu.make_async_copy(bufs[p], tail, ssems[p]).wait()

    return gather_kernel(table, idx3)


_E_BLK = 12800
_NB = _EH // _E_BLK
_OUT_ROWS = _E_BLK // 128


def _mlp_body(xi_ref, xj_ref, eat_ref, wa_ref, wb_ref, wc_ref, b1_ref, w2_ref,
              b2_ref, out_ref):
    h = jnp.dot(xi_ref[...], wa_ref[...], preferred_element_type=jnp.float32)
    h += jnp.dot(xj_ref[...], wb_ref[...], preferred_element_type=jnp.float32)
    h += jax.lax.dot_general(
        eat_ref[...], wc_ref[...], (((0,), (0,)), ((), ())),
        preferred_element_type=jnp.float32)
    h += b1_ref[...]
    h = jnp.where(h >= 0, h, NEG_SLOPE * h)
    res = jnp.sum(h * w2_ref[...], axis=1) + b2_ref[0, 0]
    out_ref[...] = res.reshape(1, _OUT_ROWS, 128)


def _tc_mlp(g, ea, wa, wb, wc, b1, w2, b2):
    return pl.pallas_call(
        _mlp_body,
        grid=(_NB,),
        in_specs=[
            pl.BlockSpec((_E_BLK, DIM_NODE), lambda i: (i, 0)),
            pl.BlockSpec((_E_BLK, DIM_NODE), lambda i: (i + _NB, 0)),
            pl.BlockSpec((DIM_EDGE, _E_BLK), lambda i: (0, i)),
            pl.BlockSpec((DIM_NODE, HID), lambda i: (0, 0)),
            pl.BlockSpec((DIM_NODE, HID), lambda i: (0, 0)),
            pl.BlockSpec((DIM_EDGE, HID), lambda i: (0, 0)),
            pl.BlockSpec((1, HID), lambda i: (0, 0)),
            pl.BlockSpec((1, HID), lambda i: (0, 0)),
            pl.BlockSpec((1, 1), lambda i: (0, 0)),
        ],
        out_specs=pl.BlockSpec((1, _OUT_ROWS, 128), lambda i: (i, 0, 0)),
        out_shape=jax.ShapeDtypeStruct((_NB, _OUT_ROWS, 128), jnp.float32),
    )(g, g, ea, wa, wb, wc, b1, w2, b2)


def kernel(x, edge_index, edge_attr, W1, b1, W2, b2):
    src = edge_index[0, :].astype(jnp.int32)
    dst = edge_index[1, :].astype(jnp.int32)
    ea_t = edge_attr.T
    pad = jnp.arange(_B - 2 * _EH, dtype=jnp.int32) % N_NODES

    w1t = W1.T
    wa = w1t[:DIM_NODE]
    wb = w1t[DIM_NODE:2 * DIM_NODE]
    wc = w1t[2 * DIM_NODE:]
    b1r = b1.reshape(1, HID)
    w2r = W2.reshape(1, HID)
    b2r = b2.reshape(1, 1)

    outs = []
    for k in range(_S):
        sl = slice(k * _EH, (k + 1) * _EH)
        idx3 = (jnp.concatenate([src[sl], dst[sl], pad])
                .reshape(_NCHUNK, _NW, _CHUNK)
                .transpose(1, 0, 2))
        g = _sc_gather(x, idx3)
        outs.append(_tc_mlp(g, ea_t[:, sl], wa, wb, wc, b1r, w2r, b2r))
    return jnp.concatenate(outs, axis=0).reshape(N_EDGES, 1)
